# baseline (speedup 1.0000x reference)
import jax
import jax.numpy as jnp
from jax import lax
from jax.experimental import pallas as pl
from jax.experimental.pallas import tpu as pltpu
from jax.experimental.pallas import tpu_sc as plsc

_W = 64
_NW = 32


def kernel(edge_index, edge_attr, init):
    n_edges, d_edge = edge_attr.shape
    n_nodes, d_feat = init.shape
    d_out = d_edge + 2 * d_feat
    nb = n_edges // _W
    assert n_edges % _W == 0
    assert nb // _NW >= 2
    lane = d_edge
    keep = d_feat - lane
    trips = (nb + _NW - 1) // _NW
    trips += trips % 2

    idx = edge_index.astype(jnp.int32)
    src = idx[0].reshape(nb, 1, _W)
    dst = idx[1].reshape(nb, 1, _W)
    t0 = jnp.pad(init[:, :keep], ((0, 0), (lane, 0)))
    t2 = jnp.pad(init[:, keep:], ((0, 0), (0, keep)))

    mesh = plsc.VectorSubcoreMesh(core_axis_name="c", subcore_axis_name="s")

    f32 = jnp.float32

    @pl.kernel(
        out_type=jax.ShapeDtypeStruct((n_edges, d_out), f32),
        mesh=mesh,
        scratch_types=[
            pltpu.VMEM((1, _W), jnp.int32),
            pltpu.VMEM((1, _W), jnp.int32),
            pltpu.VMEM((1, _W), jnp.int32),
            pltpu.VMEM((1, _W), jnp.int32),
            pltpu.VMEM((_W, d_feat), f32),
            pltpu.VMEM((_W, d_feat), f32),
            pltpu.VMEM((_W, d_feat), f32),
            pltpu.VMEM((_W, d_feat), f32),
            pltpu.VMEM((_W, d_edge), f32),
            pltpu.VMEM((_W, d_edge), f32),
            pltpu.VMEM((_W, d_out), f32),
            pltpu.VMEM((_W, d_out), f32),
        ]
        + [pltpu.SemaphoreType.DMA] * 16,
    )
    def k(t0_hbm, t2_hbm, src_hbm, dst_hbm, attr_hbm, o_hbm, *sc):
        is_ = sc[0:2]
        id_ = sc[2:4]
        ru_ = sc[4:6]
        rt_ = sc[6:8]
        at_ = sc[8:10]
        o_ = sc[10:12]
        s_is = sc[12:14]
        s_id = sc[14:16]
        s_g0 = sc[16:18]
        s_g1 = sc[18:20]
        s_g2 = sc[20:22]
        s_g3 = sc[22:24]
        s_at = sc[24:26]
        s_out = sc[26:28]

        wid = lax.axis_index("s") * 2 + lax.axis_index("c")

        def issue_idx(b, p):
            pltpu.async_copy(src_hbm.at[b], is_[p], s_is[p])
            pltpu.async_copy(dst_hbm.at[b], id_[p], s_id[p])

        def wait_idx(p):
            pltpu.make_async_copy(src_hbm.at[0], is_[p], s_is[p]).wait()
            pltpu.make_async_copy(dst_hbm.at[0], id_[p], s_id[p]).wait()

        def issue_gather(b, p):
            pltpu.async_copy(
                t0_hbm.at[is_[p].at[0]], o_[p].at[:, pl.ds(0, d_feat)], s_g0[p]
            )
            pltpu.async_copy(
                t0_hbm.at[id_[p].at[0]],
                o_[p].at[:, pl.ds(d_feat, d_feat)],
                s_g1[p],
            )
            pltpu.async_copy(t2_hbm.at[is_[p].at[0]], ru_[p], s_g2[p])
            pltpu.async_copy(t2_hbm.at[id_[p].at[0]], rt_[p], s_g3[p])
            pltpu.async_copy(attr_hbm.at[pl.ds(b * _W, _W)], at_[p], s_at[p])

        def wait_gather(p):
            pltpu.make_async_copy(
                t0_hbm.at[is_[p].at[0]], o_[p].at[:, pl.ds(0, d_feat)], s_g0[p]
            ).wait()
            pltpu.make_async_copy(
                t0_hbm.at[id_[p].at[0]],
                o_[p].at[:, pl.ds(d_feat, d_feat)],
                s_g1[p],
            ).wait()
            pltpu.make_async_copy(t2_hbm.at[is_[p].at[0]], ru_[p], s_g2[p]).wait()
            pltpu.make_async_copy(t2_hbm.at[id_[p].at[0]], rt_[p], s_g3[p]).wait()
            pltpu.make_async_copy(
                attr_hbm.at[pl.ds(0, _W)], at_[p], s_at[p]
            ).wait()

        def issue_out(b, p):
            pltpu.async_copy(o_[p], o_hbm.at[pl.ds(b * _W, _W)], s_out[p])

        def wait_out(p):
            pltpu.make_async_copy(
                o_[p], o_hbm.at[pl.ds(0, _W)], s_out[p]
            ).wait()

        def assemble(p):
            at_v, ru_v, rt_v, o_v = at_[p], ru_[p], rt_[p], o_[p]

            @pl.loop(0, _W)
            def _(i):
                a = at_v[i, pl.ds(0, lane)]
                u = ru_v[i, pl.ds(0, lane)]
                t = rt_v[i, pl.ds(0, lane)]
                o_v[i, pl.ds(0, lane)] = a
                o_v[i, pl.ds(d_feat, lane)] = u
                o_v[i, pl.ds(2 * d_feat, lane)] = t

        b0 = wid
        b1 = wid + _NW

        @pl.when(b0 < nb)
        def _():
            issue_idx(b0, 0)

        @pl.when(b1 < nb)
        def _():
            issue_idx(b1, 1)

        @pl.when(b0 < nb)
        def _():
            wait_idx(0)
            issue_gather(b0, 0)

        def trip(t, p):
            b = wid + t * _NW
            bn = b + _NW
            bnn = b + 2 * _NW
            q = 1 - p

            @pl.when(b < nb)
            def _():
                @pl.when(bn < nb)
                def _():
                    wait_idx(q)

                    @pl.when(t >= 1)
                    def _():
                        wait_out(q)

                    issue_gather(bn, q)

                wait_gather(p)

                @pl.when(bnn < nb)
                def _():
                    issue_idx(bnn, p)

                assemble(p)
                issue_out(b, p)

        @pl.loop(0, trips, step=2)
        def _(t):
            trip(t, 0)
            trip(t + 1, 1)

        wait_out(0)
        wait_out(1)

    return k(t0, t2, src, dst, edge_attr)


# speedup vs baseline: 1.1320x; 1.1320x over previous
import jax
import jax.numpy as jnp
from jax import lax
from jax.experimental import pallas as pl
from jax.experimental.pallas import tpu as pltpu
from jax.experimental.pallas import tpu_sc as plsc

_W = 64
_NW = 32


def kernel(edge_index, edge_attr, init):
    n_edges, d_edge = edge_attr.shape
    n_nodes, d_feat = init.shape
    d_out = d_edge + 2 * d_feat
    nb = n_edges // _W
    assert n_edges % _W == 0
    assert nb // _NW >= 2
    lane = d_edge
    spf = d_feat // lane
    trips = (nb + _NW - 1) // _NW
    trips += trips % 2

    idx = edge_index.astype(jnp.int32)
    src = idx[0].reshape(nb, 1, _W)
    dst = idx[1].reshape(nb, 1, _W)
    cidx = jnp.concatenate([src, dst], axis=2)

    mesh = plsc.VectorSubcoreMesh(core_axis_name="c", subcore_axis_name="s")

    f32 = jnp.float32

    @pl.kernel(
        out_type=jax.ShapeDtypeStruct((n_edges, d_out), f32),
        mesh=mesh,
        scratch_types=[
            pltpu.VMEM((1, 2 * _W), jnp.int32),
            pltpu.VMEM((1, 2 * _W), jnp.int32),
            pltpu.VMEM((2 * _W, d_feat), f32),
            pltpu.VMEM((2 * _W, d_feat), f32),
            pltpu.VMEM((_W, d_edge), f32),
            pltpu.VMEM((_W, d_edge), f32),
            pltpu.VMEM((_W, d_out), f32),
            pltpu.VMEM((_W, d_out), f32),
        ]
        + [pltpu.SemaphoreType.DMA] * 8,
    )
    def k(init_hbm, cidx_hbm, attr_hbm, o_hbm, *sc):
        ix_ = sc[0:2]
        rg_ = sc[2:4]
        at_ = sc[4:6]
        o_ = sc[6:8]
        s_ix = sc[8:10]
        s_g = sc[10:12]
        s_at = sc[12:14]
        s_out = sc[14:16]

        wid = lax.axis_index("s") * 2 + lax.axis_index("c")

        def issue_idx(b, p):
            pltpu.async_copy(cidx_hbm.at[b], ix_[p], s_ix[p])

        def wait_idx(p):
            pltpu.make_async_copy(cidx_hbm.at[0], ix_[p], s_ix[p]).wait()

        def issue_gather(b, p):
            pltpu.async_copy(init_hbm.at[ix_[p].at[0]], rg_[p], s_g[p])
            pltpu.async_copy(attr_hbm.at[pl.ds(b * _W, _W)], at_[p], s_at[p])

        def wait_gather(p):
            pltpu.make_async_copy(
                init_hbm.at[ix_[p].at[0]], rg_[p], s_g[p]
            ).wait()
            pltpu.make_async_copy(
                attr_hbm.at[pl.ds(0, _W)], at_[p], s_at[p]
            ).wait()

        def issue_out(b, p):
            pltpu.async_copy(o_[p], o_hbm.at[pl.ds(b * _W, _W)], s_out[p])

        def wait_out(p):
            pltpu.make_async_copy(
                o_[p], o_hbm.at[pl.ds(0, _W)], s_out[p]
            ).wait()

        def assemble(p):
            at_v, rg_v, o_v = at_[p], rg_[p], o_[p]

            @pl.loop(0, _W)
            def _(i):
                vals = [at_v[i, pl.ds(0, lane)]]
                vals += [rg_v[i, pl.ds(kk * lane, lane)] for kk in range(spf)]
                vals += [
                    rg_v[_W + i, pl.ds(kk * lane, lane)] for kk in range(spf)
                ]
                for j, v in enumerate(vals):
                    o_v[i, pl.ds(j * lane, lane)] = v

        b0 = wid
        b1 = wid + _NW

        @pl.when(b0 < nb)
        def _():
            issue_idx(b0, 0)

        @pl.when(b1 < nb)
        def _():
            issue_idx(b1, 1)

        @pl.when(b0 < nb)
        def _():
            wait_idx(0)
            issue_gather(b0, 0)

        def trip(t, p):
            b = wid + t * _NW
            bn = b + _NW
            bnn = b + 2 * _NW
            q = 1 - p

            @pl.when(b < nb)
            def _():
                @pl.when(bn < nb)
                def _():
                    wait_idx(q)
                    issue_gather(bn, q)

                wait_gather(p)

                @pl.when(bnn < nb)
                def _():
                    issue_idx(bnn, p)

                @pl.when(t >= 2)
                def _():
                    wait_out(p)

                assemble(p)
                issue_out(b, p)

        @pl.loop(0, trips, step=2)
        def _(t):
            trip(t, 0)
            trip(t + 1, 1)

        wait_out(0)
        wait_out(1)

    return k(init, cidx, edge_attr)
